# baseline (speedup 1.0000x reference)
import jax
import jax.numpy as jnp
from jax import lax
from jax.experimental import pallas as pl
from jax.experimental.pallas import tpu as pltpu
from jax.experimental.pallas import tpu_sc as plsc

_B = 16
_P = 1024
_K = 20
_N = _B * _P
_EPS = 1e-5


_NW = 32
_GCH = 128


def _sc_gather(table, idx):
    (E,) = idx.shape
    R, D = table.shape
    per_w = E // _NW
    steps = per_w // _GCH
    assert per_w * _NW == E and steps * _GCH == per_w
    mesh = plsc.VectorSubcoreMesh(core_axis_name="c", subcore_axis_name="s")

    def body(table_h, idx_h, out_h, idx_v, rows_v, sem):
        wid = lax.axis_index("s") * 2 + lax.axis_index("c")
        base = wid * per_w
        pltpu.sync_copy(idx_h.at[pl.ds(base, per_w)], idx_v)

        def step(g, carry):
            pltpu.async_copy(
                table_h.at[idx_v.at[pl.ds(g * _GCH, _GCH)]], rows_v, sem
            ).wait()
            pltpu.sync_copy(rows_v, out_h.at[pl.ds(base + g * _GCH, _GCH)])
            return carry

        lax.fori_loop(0, steps, step, 0)

    return pl.kernel(
        body,
        out_type=jax.ShapeDtypeStruct((E, D), jnp.float32),
        mesh=mesh,
        scratch_types=[
            pltpu.VMEM((per_w,), jnp.int32),
            pltpu.VMEM((_GCH, D), jnp.float32),
            pltpu.SemaphoreType.DMA,
        ],
    )(table, idx)


def _sc_gather_xyz(pos4, lidx):
    (E,) = lidx.shape
    Bq, Pq, _ = pos4.shape
    per_w = E // _NW
    mesh = plsc.VectorSubcoreMesh(core_axis_name="c", subcore_axis_name="s")

    pos_flat = pos4.reshape(Bq, Pq * 4)

    def body(pos_h, idx_h, out_h, pos_v, idx_v, xyz_v):
        wid = lax.axis_index("s") * 2 + lax.axis_index("c")
        g = wid // (_NW // Bq)
        base = wid * per_w
        pltpu.sync_copy(pos_h.at[g], pos_v)
        pltpu.sync_copy(idx_h.at[pl.ds(base, per_w)], idx_v)

        def step(t, carry):
            ii = idx_v[pl.ds(t * 16, 16)] * 4
            for c in range(3):
                vals = plsc.load_gather(pos_v, [ii + c])
                xyz_v[c, pl.ds(t * 16, 16)] = vals
            return carry

        lax.fori_loop(0, per_w // 16, step, 0)
        pltpu.sync_copy(xyz_v, out_h.at[:, pl.ds(base, per_w)])

    return pl.kernel(
        body,
        out_type=jax.ShapeDtypeStruct((4, E), jnp.float32),
        mesh=mesh,
        compiler_params=pltpu.CompilerParams(needs_layout_passes=False),
        scratch_types=[
            pltpu.VMEM((Pq * 4,), jnp.float32),
            pltpu.VMEM((per_w,), jnp.int32),
            pltpu.VMEM((4, per_w), jnp.float32),
        ],
    )(pos_flat, lidx)


def _edge1_stats_pallas(xjp, x, W1aT, b1a):
    E = xjp.shape[1]
    G = x.shape[0]
    EG = E // G

    def body(xjp_ref, xt_ref, w_ref, b_ref, o_ref):
        xt = xt_ref[0]
        xi = jnp.broadcast_to(xt[:, None, :], (3, _K, _P)).reshape(3, EG)
        xj = xjp_ref[:3]
        f1t = jnp.concatenate([xi, xj - xi], axis=0)
        h = jnp.dot(w_ref[...], f1t, preferred_element_type=jnp.float32)
        h = h + b_ref[...]
        o_ref[0, 0, :] = jnp.sum(h, axis=1)
        o_ref[0, 1, :] = jnp.sum(h * h, axis=1)

    return pl.pallas_call(
        body,
        grid=(G,),
        in_specs=[
            pl.BlockSpec((4, EG), lambda g: (0, g)),
            pl.BlockSpec((1, 3, _P), lambda g: (g, 0, 0)),
            pl.BlockSpec((64, 6), lambda g: (0, 0)),
            pl.BlockSpec((64, 1), lambda g: (0, 0)),
        ],
        out_specs=pl.BlockSpec((1, 2, 64), lambda g: (g, 0, 0)),
        out_shape=jax.ShapeDtypeStruct((G, 2, 64), jnp.float32),
    )(xjp, jnp.transpose(x, (0, 2, 1)), W1aT, b1a.reshape(64, 1))


def _edge1_apply_pallas(xjp, x, W1aT, b1a, bn, W1bT, b1b):
    E = xjp.shape[1]
    G = x.shape[0]
    EG = E // G

    def body(xjp_ref, xt_ref, wa_ref, ba_ref, bn_ref, wb_ref, bb_ref, o_ref):
        xt = xt_ref[0]
        xi = jnp.broadcast_to(xt[:, None, :], (3, _K, _P)).reshape(3, EG)
        xj = xjp_ref[:3]
        f1t = jnp.concatenate([xi, xj - xi], axis=0)
        h = jnp.dot(wa_ref[...], f1t, preferred_element_type=jnp.float32)
        h = h + ba_ref[...]
        mu = bn_ref[0, 0, :].reshape(64, 1)
        sv = bn_ref[0, 1, :].reshape(64, 1)
        ga = bn_ref[0, 2, :].reshape(64, 1)
        be = bn_ref[0, 3, :].reshape(64, 1)
        h = (h - mu) / sv * ga + be
        h = jnp.maximum(h, 0.0)
        h2 = jnp.dot(wb_ref[...], h, preferred_element_type=jnp.float32)
        h2 = h2 + bb_ref[...]
        o_ref[0] = jnp.max(h2.reshape(64, _K, _P), axis=1)

    return pl.pallas_call(
        body,
        grid=(G,),
        in_specs=[
            pl.BlockSpec((4, EG), lambda g: (0, g)),
            pl.BlockSpec((1, 3, _P), lambda g: (g, 0, 0)),
            pl.BlockSpec((64, 6), lambda g: (0, 0)),
            pl.BlockSpec((64, 1), lambda g: (0, 0)),
            pl.BlockSpec((1, 4, 64), lambda g: (g // _B, 0, 0)),
            pl.BlockSpec((64, 64), lambda g: (0, 0)),
            pl.BlockSpec((64, 1), lambda g: (0, 0)),
        ],
        out_specs=pl.BlockSpec((1, 64, _P), lambda g: (g, 0, 0)),
        out_shape=jax.ShapeDtypeStruct((G, 64, _P), jnp.float32),
    )(xjp, jnp.transpose(x, (0, 2, 1)), W1aT, b1a.reshape(64, 1), bn,
      W1bT, b1b.reshape(64, 1))


def _q_pallas(x1, W2b):
    def body(x_ref, w_ref, o_ref):
        o_ref[0] = jnp.dot(x_ref[0], w_ref[...],
                           preferred_element_type=jnp.float32)

    G = x1.shape[0]
    return pl.pallas_call(
        body,
        grid=(G,),
        in_specs=[pl.BlockSpec((1, _P, 64), lambda g: (g, 0, 0)),
                  pl.BlockSpec((64, 128), lambda g: (0, 0))],
        out_specs=pl.BlockSpec((1, _P, 128), lambda g: (g, 0, 0)),
        out_shape=jax.ShapeDtypeStruct((G, _P, 128), jnp.float32),
    )(x1, W2b)


def _tail_pallas(x1, qg, W2d, b2, Wl, bl):
    EG = _P * _K

    def body(x_ref, qg_ref, wd_ref, b2_ref, wl_ref, bl_ref, o_ref):
        x1b = x_ref[0]
        p = jnp.dot(x1b, wd_ref[...], preferred_element_type=jnp.float32)
        p = p + b2_ref[...]
        mq = jnp.max(qg_ref[...].reshape(_P, _K, 128), axis=1)
        x2 = p + mq
        cat = jnp.concatenate([x1b, x2], axis=1)
        hp = jnp.dot(cat, wl_ref[...], preferred_element_type=jnp.float32)
        hp = hp + bl_ref[...]
        o_ref[0] = jnp.max(hp, axis=0, keepdims=True)

    G = x1.shape[0]
    return pl.pallas_call(
        body,
        grid=(G,),
        in_specs=[
            pl.BlockSpec((1, _P, 64), lambda g: (g, 0, 0)),
            pl.BlockSpec((EG, 128), lambda g: (g, 0)),
            pl.BlockSpec((64, 128), lambda g: (0, 0)),
            pl.BlockSpec((1, 128), lambda g: (0, 0)),
            pl.BlockSpec((192, 128), lambda g: (0, 0)),
            pl.BlockSpec((1, 128), lambda g: (0, 0)),
        ],
        out_specs=pl.BlockSpec((1, 1, 128), lambda g: (g, 0, 0)),
        out_shape=jax.ShapeDtypeStruct((G, 1, 128), jnp.float32),
    )(x1, qg, W2d, b2.reshape(1, 128), Wl, bl.reshape(1, 128))[:, 0, :]


def _head_pallas(h, W3a, b3a, W3b, b3b):
    def body(h_ref, wa_ref, ba_ref, wb_ref, bb_ref, o_ref):
        t = jnp.maximum(
            jnp.dot(h_ref[...], wa_ref[...], preferred_element_type=jnp.float32)
            + ba_ref[...],
            0.0,
        )
        o_ref[...] = (
            jnp.dot(t, wb_ref[...], preferred_element_type=jnp.float32) + bb_ref[...]
        )

    return pl.pallas_call(
        body,
        out_shape=jax.ShapeDtypeStruct((h.shape[0], 32), jnp.float32),
    )(h, W3a, b3a.reshape(1, -1), W3b, b3b.reshape(1, -1))


def _knn_idx(x):
    Bq, Pq, d = x.shape
    xt = jnp.transpose(x, (0, 2, 1))

    def body(x_ref, xt_ref, o_ref):
        xx = x_ref[0]
        xxt = xt_ref[0]
        sq = jnp.sum(xx * xx, axis=1)
        cross = jnp.dot(xx, xxt, preferred_element_type=jnp.float32)
        d2 = sq[:, None] + sq[None, :] - 2.0 * cross
        colio = jax.lax.broadcasted_iota(jnp.int32, (Pq, Pq), 1)
        for k in range(_K):
            m = jnp.min(d2, axis=1, keepdims=True)
            idx = jnp.min(jnp.where(d2 <= m, colio, Pq), axis=1)
            o_ref[0, k, :] = idx
            d2 = jnp.where(colio == idx[:, None], jnp.inf, d2)

    out = pl.pallas_call(
        body,
        grid=(Bq,),
        in_specs=[pl.BlockSpec((1, Pq, d), lambda b: (b, 0, 0)),
                  pl.BlockSpec((1, d, Pq), lambda b: (b, 0, 0))],
        out_specs=pl.BlockSpec((1, 32, Pq), lambda b: (b, 0, 0)),
        out_shape=jax.ShapeDtypeStruct((Bq, 32, Pq), jnp.int32),
    )(x, xt)
    return jnp.transpose(out[:, :_K, :], (0, 2, 1))


def kernel(pos_1, pos_2, batch, W1a, b1a, g1a, be1a, W1b, b1b, W2, b2, Wl, bl, W3a, b3a, W3b, b3b):
    G = 2 * _B
    x = jnp.concatenate([pos_1.reshape(_B, _P, 3),
                         pos_2.reshape(_B, _P, 3)], axis=0)
    idx = _knn_idx(x)

    offs = (jnp.arange(G, dtype=jnp.int32) * _P)[:, None, None]
    pos4 = jnp.pad(x, ((0, 0), (0, 0), (0, 1)))
    lidx = jnp.transpose(idx, (0, 2, 1)).reshape(-1)
    xjp = _sc_gather_xyz(pos4, lidx)
    W1aT = jnp.transpose(W1a)
    sums = _edge1_stats_pallas(xjp, x, W1aT, b1a)
    S = jnp.stack([jnp.sum(sums[:_B], axis=0),
                   jnp.sum(sums[_B:], axis=0)])
    nE = float(_N * _K)
    mu = S[:, 0] / nE
    var = S[:, 1] / nE - mu * mu
    sv = jnp.sqrt(var + _EPS)
    bn = jnp.stack([mu, sv,
                    jnp.broadcast_to(g1a, (2, 64)),
                    jnp.broadcast_to(be1a, (2, 64))], axis=1)
    x1t = _edge1_apply_pallas(xjp, x, W1aT, b1a, bn, jnp.transpose(W1b), b1b)
    x1 = jnp.transpose(x1t, (0, 2, 1))

    idx2 = _knn_idx(x1)
    q = _q_pallas(x1, W2[64:])
    gidx2 = (idx2 + offs).reshape(-1)
    qg = _sc_gather(q.reshape(G * _P, 128), gidx2)
    h = _tail_pallas(x1, qg, W2[:64] - W2[64:], b2, Wl, bl)
    c = _head_pallas(h, W3a, b3a, W3b, b3b)
    return (h[:_B], h[_B:], c[:_B], c[_B:])


# speedup vs baseline: 1.0386x; 1.0386x over previous
import jax
import jax.numpy as jnp
from jax import lax
from jax.experimental import pallas as pl
from jax.experimental.pallas import tpu as pltpu
from jax.experimental.pallas import tpu_sc as plsc

_B = 16
_P = 1024
_K = 20
_N = _B * _P
_EPS = 1e-5


_NW = 32
_GCH = 512


def _sc_gather(table, idx):
    (E,) = idx.shape
    R, D = table.shape
    per_w = E // _NW
    steps = per_w // _GCH
    assert per_w * _NW == E and steps * _GCH == per_w
    mesh = plsc.VectorSubcoreMesh(core_axis_name="c", subcore_axis_name="s")

    def body(table_h, idx_h, out_h, idx_v, rows_v, sem):
        wid = lax.axis_index("s") * 2 + lax.axis_index("c")
        base = wid * per_w
        pltpu.sync_copy(idx_h.at[pl.ds(base, per_w)], idx_v)

        def step(g, carry):
            pltpu.async_copy(
                table_h.at[idx_v.at[pl.ds(g * _GCH, _GCH)]], rows_v, sem
            ).wait()
            pltpu.sync_copy(rows_v, out_h.at[pl.ds(base + g * _GCH, _GCH)])
            return carry

        lax.fori_loop(0, steps, step, 0)

    return pl.kernel(
        body,
        out_type=jax.ShapeDtypeStruct((E, D), jnp.float32),
        mesh=mesh,
        scratch_types=[
            pltpu.VMEM((per_w,), jnp.int32),
            pltpu.VMEM((_GCH, D), jnp.float32),
            pltpu.SemaphoreType.DMA,
        ],
    )(table, idx)


def _sc_gather_xyz(pos4, lidx):
    (E,) = lidx.shape
    Bq, Pq, _ = pos4.shape
    per_w = E // _NW
    mesh = plsc.VectorSubcoreMesh(core_axis_name="c", subcore_axis_name="s")

    pos_flat = pos4.reshape(Bq, Pq * 4)

    def body(pos_h, idx_h, out_h, pos_v, idx_v, xyz_v):
        wid = lax.axis_index("s") * 2 + lax.axis_index("c")
        g = wid // (_NW // Bq)
        base = wid * per_w
        pltpu.sync_copy(pos_h.at[g], pos_v)
        pltpu.sync_copy(idx_h.at[pl.ds(base, per_w)], idx_v)

        def step(t, carry):
            ii = idx_v[pl.ds(t * 16, 16)] * 4
            for c in range(3):
                vals = plsc.load_gather(pos_v, [ii + c])
                xyz_v[c, pl.ds(t * 16, 16)] = vals
            return carry

        lax.fori_loop(0, per_w // 16, step, 0)
        pltpu.sync_copy(xyz_v, out_h.at[:, pl.ds(base, per_w)])

    return pl.kernel(
        body,
        out_type=jax.ShapeDtypeStruct((4, E), jnp.float32),
        mesh=mesh,
        compiler_params=pltpu.CompilerParams(needs_layout_passes=False),
        scratch_types=[
            pltpu.VMEM((Pq * 4,), jnp.float32),
            pltpu.VMEM((per_w,), jnp.int32),
            pltpu.VMEM((4, per_w), jnp.float32),
        ],
    )(pos_flat, lidx)


def _edge1_stats_pallas(xjp, x, W1aT, b1a):
    E = xjp.shape[1]
    G = x.shape[0]
    EG = E // G

    def body(xjp_ref, xt_ref, w_ref, b_ref, o_ref):
        xt = xt_ref[0]
        xi = jnp.broadcast_to(xt[:, None, :], (3, _K, _P)).reshape(3, EG)
        xj = xjp_ref[:3]
        f1t = jnp.concatenate([xi, xj - xi], axis=0)
        h = jnp.dot(w_ref[...], f1t, preferred_element_type=jnp.float32)
        h = h + b_ref[...]
        o_ref[0, 0, :] = jnp.sum(h, axis=1)
        o_ref[0, 1, :] = jnp.sum(h * h, axis=1)

    return pl.pallas_call(
        body,
        grid=(G,),
        in_specs=[
            pl.BlockSpec((4, EG), lambda g: (0, g)),
            pl.BlockSpec((1, 3, _P), lambda g: (g, 0, 0)),
            pl.BlockSpec((64, 6), lambda g: (0, 0)),
            pl.BlockSpec((64, 1), lambda g: (0, 0)),
        ],
        out_specs=pl.BlockSpec((1, 2, 64), lambda g: (g, 0, 0)),
        out_shape=jax.ShapeDtypeStruct((G, 2, 64), jnp.float32),
    )(xjp, jnp.transpose(x, (0, 2, 1)), W1aT, b1a.reshape(64, 1))


def _edge1_apply_pallas(xjp, x, W1aT, b1a, bn, W1bT, b1b):
    E = xjp.shape[1]
    G = x.shape[0]
    EG = E // G

    def body(xjp_ref, xt_ref, wa_ref, ba_ref, bn_ref, wb_ref, bb_ref, o_ref):
        xt = xt_ref[0]
        xi = jnp.broadcast_to(xt[:, None, :], (3, _K, _P)).reshape(3, EG)
        xj = xjp_ref[:3]
        f1t = jnp.concatenate([xi, xj - xi], axis=0)
        h = jnp.dot(wa_ref[...], f1t, preferred_element_type=jnp.float32)
        h = h + ba_ref[...]
        mu = bn_ref[0, 0, :].reshape(64, 1)
        sv = bn_ref[0, 1, :].reshape(64, 1)
        ga = bn_ref[0, 2, :].reshape(64, 1)
        be = bn_ref[0, 3, :].reshape(64, 1)
        h = (h - mu) / sv * ga + be
        h = jnp.maximum(h, 0.0)
        h2 = jnp.dot(wb_ref[...], h, preferred_element_type=jnp.float32)
        h2 = h2 + bb_ref[...]
        o_ref[0] = jnp.max(h2.reshape(64, _K, _P), axis=1)

    return pl.pallas_call(
        body,
        grid=(G,),
        in_specs=[
            pl.BlockSpec((4, EG), lambda g: (0, g)),
            pl.BlockSpec((1, 3, _P), lambda g: (g, 0, 0)),
            pl.BlockSpec((64, 6), lambda g: (0, 0)),
            pl.BlockSpec((64, 1), lambda g: (0, 0)),
            pl.BlockSpec((1, 4, 64), lambda g: (g // _B, 0, 0)),
            pl.BlockSpec((64, 64), lambda g: (0, 0)),
            pl.BlockSpec((64, 1), lambda g: (0, 0)),
        ],
        out_specs=pl.BlockSpec((1, 64, _P), lambda g: (g, 0, 0)),
        out_shape=jax.ShapeDtypeStruct((G, 64, _P), jnp.float32),
    )(xjp, jnp.transpose(x, (0, 2, 1)), W1aT, b1a.reshape(64, 1), bn,
      W1bT, b1b.reshape(64, 1))


def _q_pallas(x1, W2b):
    def body(x_ref, w_ref, o_ref):
        o_ref[0] = jnp.dot(x_ref[0], w_ref[...],
                           preferred_element_type=jnp.float32)

    G = x1.shape[0]
    return pl.pallas_call(
        body,
        grid=(G,),
        in_specs=[pl.BlockSpec((1, _P, 64), lambda g: (g, 0, 0)),
                  pl.BlockSpec((64, 128), lambda g: (0, 0))],
        out_specs=pl.BlockSpec((1, _P, 128), lambda g: (g, 0, 0)),
        out_shape=jax.ShapeDtypeStruct((G, _P, 128), jnp.float32),
    )(x1, W2b)


def _tail_pallas(x1, qg, W2d, b2, Wl, bl):
    EG = _P * _K

    def body(x_ref, qg_ref, wd_ref, b2_ref, wl_ref, bl_ref, o_ref):
        x1b = x_ref[0]
        p = jnp.dot(x1b, wd_ref[...], preferred_element_type=jnp.float32)
        p = p + b2_ref[...]
        mq = jnp.max(qg_ref[...].reshape(_P, _K, 128), axis=1)
        x2 = p + mq
        cat = jnp.concatenate([x1b, x2], axis=1)
        hp = jnp.dot(cat, wl_ref[...], preferred_element_type=jnp.float32)
        hp = hp + bl_ref[...]
        o_ref[0] = jnp.max(hp, axis=0, keepdims=True)

    G = x1.shape[0]
    return pl.pallas_call(
        body,
        grid=(G,),
        in_specs=[
            pl.BlockSpec((1, _P, 64), lambda g: (g, 0, 0)),
            pl.BlockSpec((EG, 128), lambda g: (g, 0)),
            pl.BlockSpec((64, 128), lambda g: (0, 0)),
            pl.BlockSpec((1, 128), lambda g: (0, 0)),
            pl.BlockSpec((192, 128), lambda g: (0, 0)),
            pl.BlockSpec((1, 128), lambda g: (0, 0)),
        ],
        out_specs=pl.BlockSpec((1, 1, 128), lambda g: (g, 0, 0)),
        out_shape=jax.ShapeDtypeStruct((G, 1, 128), jnp.float32),
    )(x1, qg, W2d, b2.reshape(1, 128), Wl, bl.reshape(1, 128))[:, 0, :]


def _head_pallas(h, W3a, b3a, W3b, b3b):
    def body(h_ref, wa_ref, ba_ref, wb_ref, bb_ref, o_ref):
        t = jnp.maximum(
            jnp.dot(h_ref[...], wa_ref[...], preferred_element_type=jnp.float32)
            + ba_ref[...],
            0.0,
        )
        o_ref[...] = (
            jnp.dot(t, wb_ref[...], preferred_element_type=jnp.float32) + bb_ref[...]
        )

    return pl.pallas_call(
        body,
        out_shape=jax.ShapeDtypeStruct((h.shape[0], 32), jnp.float32),
    )(h, W3a, b3a.reshape(1, -1), W3b, b3b.reshape(1, -1))


def _knn_idx(x):
    Bq, Pq, d = x.shape
    xt = jnp.transpose(x, (0, 2, 1))

    def body(x_ref, xt_ref, o_ref):
        xx = x_ref[0]
        xxt = xt_ref[0]
        sq = jnp.sum(xx * xx, axis=1)
        cross = jnp.dot(xx, xxt, preferred_element_type=jnp.float32)
        d2 = sq[:, None] + sq[None, :] - 2.0 * cross
        colio = jax.lax.broadcasted_iota(jnp.int32, (Pq, Pq), 1)
        for k in range(_K):
            m = jnp.min(d2, axis=1, keepdims=True)
            idx = jnp.min(jnp.where(d2 <= m, colio, Pq), axis=1)
            o_ref[0, k, :] = idx
            d2 = jnp.where(colio == idx[:, None], jnp.inf, d2)

    out = pl.pallas_call(
        body,
        grid=(Bq,),
        in_specs=[pl.BlockSpec((1, Pq, d), lambda b: (b, 0, 0)),
                  pl.BlockSpec((1, d, Pq), lambda b: (b, 0, 0))],
        out_specs=pl.BlockSpec((1, 32, Pq), lambda b: (b, 0, 0)),
        out_shape=jax.ShapeDtypeStruct((Bq, 32, Pq), jnp.int32),
    )(x, xt)
    return jnp.transpose(out[:, :_K, :], (0, 2, 1))


def kernel(pos_1, pos_2, batch, W1a, b1a, g1a, be1a, W1b, b1b, W2, b2, Wl, bl, W3a, b3a, W3b, b3b):
    G = 2 * _B
    x = jnp.concatenate([pos_1.reshape(_B, _P, 3),
                         pos_2.reshape(_B, _P, 3)], axis=0)
    idx = _knn_idx(x)

    offs = (jnp.arange(G, dtype=jnp.int32) * _P)[:, None, None]
    pos4 = jnp.pad(x, ((0, 0), (0, 0), (0, 1)))
    lidx = jnp.transpose(idx, (0, 2, 1)).reshape(-1)
    xjp = _sc_gather_xyz(pos4, lidx)
    W1aT = jnp.transpose(W1a)
    sums = _edge1_stats_pallas(xjp, x, W1aT, b1a)
    S = jnp.stack([jnp.sum(sums[:_B], axis=0),
                   jnp.sum(sums[_B:], axis=0)])
    nE = float(_N * _K)
    mu = S[:, 0] / nE
    var = S[:, 1] / nE - mu * mu
    sv = jnp.sqrt(var + _EPS)
    bn = jnp.stack([mu, sv,
                    jnp.broadcast_to(g1a, (2, 64)),
                    jnp.broadcast_to(be1a, (2, 64))], axis=1)
    x1t = _edge1_apply_pallas(xjp, x, W1aT, b1a, bn, jnp.transpose(W1b), b1b)
    x1 = jnp.transpose(x1t, (0, 2, 1))

    idx2 = _knn_idx(x1)
    q = _q_pallas(x1, W2[64:])
    gidx2 = (idx2 + offs).reshape(-1)
    qg = _sc_gather(q.reshape(G * _P, 128), gidx2)
    h = _tail_pallas(x1, qg, W2[:64] - W2[64:], b2, Wl, bl)
    c = _head_pallas(h, W3a, b3a, W3b, b3b)
    return (h[:_B], h[_B:], c[:_B], c[_B:])
